# Initial kernel scaffold; baseline (speedup 1.0000x reference)
import jax
import jax.numpy as jnp
from jax.experimental import pallas as pl


def kernel(xyz, features, W1, b1, W2, b2, W3, b3):
    raise NotImplementedError("write your pallas kernel here")


# speedup vs baseline: 7.4679x; 7.4679x over previous
import functools

import jax
import jax.numpy as jnp
import numpy as np
from jax import lax
from jax.experimental import pallas as pl
from jax.experimental.pallas import tpu as pltpu
from jax.experimental.pallas import tpu_sc as plsc

_B, _N, _C = 4, 8192, 16
_NP, _NS = 1024, 32
_R2 = float(np.float32(0.8 * 0.8))
_ROW = 24
_CBLK = 128


def _fps_body(xyz_ref, new_ref):
    x = xyz_ref[0, 0]
    y = xyz_ref[0, 1]
    z = xyz_ref[0, 2]
    ii = (lax.broadcasted_iota(jnp.int32, (8, 1024), 0) * 1024
          + lax.broadcasted_iota(jnp.int32, (8, 1024), 1))

    def coords(p):
        pm = ii == p
        zero = jnp.float32(0.0)
        return (jnp.sum(jnp.where(pm, x, zero)),
                jnp.sum(jnp.where(pm, y, zero)),
                jnp.sum(jnp.where(pm, z, zero)))

    def body(i, st):
        dists, p = st
        lx, ly, lz = coords(p)
        new_ref[0, 0, i - 1] = lx
        new_ref[0, 1, i - 1] = ly
        new_ref[0, 2, i - 1] = lz
        dx = x - lx
        dy = y - ly
        dz = z - lz
        d = (dx * dx + dy * dy) + dz * dz
        dists = jnp.minimum(dists, d)
        m = jnp.max(dists)
        cand = jnp.where(dists == m, ii, _N)
        return dists, jnp.min(cand)

    dists0 = jnp.full((8, 1024), 1e10, jnp.float32)
    _, p = lax.fori_loop(1, _NP, body, (dists0, jnp.int32(0)))
    lx, ly, lz = coords(p)
    new_ref[0, 0, _NP - 1] = lx
    new_ref[0, 1, _NP - 1] = ly
    new_ref[0, 2, _NP - 1] = lz


def _fps(xyz_r):
    return pl.pallas_call(
        _fps_body,
        grid=(_B,),
        in_specs=[pl.BlockSpec((1, 3, 8, 1024), lambda b: (b, 0, 0, 0))],
        out_specs=pl.BlockSpec((1, 3, _NP), lambda b: (b, 0, 0),
                               memory_space=pltpu.SMEM),
        out_shape=jax.ShapeDtypeStruct((_B, 3, _NP), jnp.float32),
    )(xyz_r)


def _bq_body(xyzp, newp, table, grouped, cntf,
             px, py, pz, cx, cy, cz, idxbuf, idx32, rows, cntb, sem):
    w = lax.axis_index("s") * 2 + lax.axis_index("c")
    b = w // 8
    c0 = (w % 8) * _CBLK

    pltpu.sync_copy(xyzp.at[b * 3 + 0], px)
    pltpu.sync_copy(xyzp.at[b * 3 + 1], py)
    pltpu.sync_copy(xyzp.at[b * 3 + 2], pz)
    pltpu.sync_copy(newp.at[b * 3 + 0], cx)
    pltpu.sync_copy(newp.at[b * 3 + 1], cy)
    pltpu.sync_copy(newp.at[b * 3 + 2], cz)

    zeros16 = jnp.zeros((16,), jnp.int32)
    lanes = lax.iota(jnp.int32, 16)
    bofs = jnp.full((16,), b * _N, jnp.int32)

    def do_centroid(j, carry):
        cg = c0 + j
        csel = jnp.full((16,), cg, jnp.int32)
        cxv = plsc.load_gather(cx, [csel])
        cyv = plsc.load_gather(cy, [csel])
        czv = plsc.load_gather(cz, [csel])
        idxbuf[pl.ds(0, 16)] = zeros16
        idxbuf[pl.ds(16, 16)] = zeros16
        idxbuf[pl.ds(32, 16)] = zeros16

        def scan(i, cnt):
            base = i * 16
            dx = px[pl.ds(base, 16)] - cxv
            dy = py[pl.ds(base, 16)] - cyv
            dz = pz[pl.ds(base, 16)] - czv
            d2 = (dx * dx + dy * dy) + dz * dz
            within = d2 < _R2

            @pl.when(cnt < _NS)
            def _():
                plsc.store_compressed(idxbuf.at[pl.ds(cnt, 16)],
                                      lanes + base, mask=within)

            inc = jnp.sum(jnp.where(within, 1, 0).astype(jnp.int32))
            return jnp.where(cnt < _NS, cnt + inc, cnt)

        cnt = lax.fori_loop(0, _N // 16, scan, jnp.int32(0))
        cnt = jnp.minimum(cnt, _NS)

        idx32[pl.ds(0, 16)] = idxbuf[pl.ds(0, 16)] + bofs
        idx32[pl.ds(16, 16)] = idxbuf[pl.ds(16, 16)] + bofs
        pltpu.async_copy(table.at[idx32], rows, sem).wait()
        pltpu.sync_copy(rows, grouped.at[b, cg])
        plsc.store_scatter(cntb, [jnp.full((16,), j, jnp.int32)],
                           jnp.full((16,), cnt.astype(jnp.float32)))
        return carry

    lax.fori_loop(0, _CBLK, do_centroid, jnp.int32(0))
    pltpu.sync_copy(cntb, cntf.at[w, 0])


_bq_cache = []


def _bq(*args):
    if not _bq_cache:
        _bq_cache.append(_make_bq())
    return _bq_cache[0](*args)


def _make_bq():
    return functools.partial(
        pl.kernel,
        mesh=plsc.VectorSubcoreMesh(core_axis_name="c", subcore_axis_name="s"),
        compiler_params=pltpu.CompilerParams(needs_layout_passes=False,
                                             use_tc_tiling_on_sc=False),
        out_type=(jax.ShapeDtypeStruct((_B, _NP, _NS, _ROW), jnp.float32),
                  jax.ShapeDtypeStruct((_B * 8, 1, _CBLK), jnp.float32)),
        scratch_types=[
        pltpu.VMEM((_N,), jnp.float32),
        pltpu.VMEM((_N,), jnp.float32),
        pltpu.VMEM((_N,), jnp.float32),
        pltpu.VMEM((_NP,), jnp.float32),
        pltpu.VMEM((_NP,), jnp.float32),
        pltpu.VMEM((_NP,), jnp.float32),
        pltpu.VMEM((48,), jnp.int32),
        pltpu.VMEM((_NS,), jnp.int32),
            pltpu.VMEM((_NS, _ROW), jnp.float32),
            pltpu.VMEM((_CBLK,), jnp.float32),
            pltpu.SemaphoreType.DMA,
        ],
    )(_bq_body)


def _mlp_body(g_ref, np_ref, cnt_ref, w1_ref, b1_ref, w2_ref, b2_ref,
              w3_ref, b3_ref, out_ref):
    xg = g_ref[0]
    lane = lax.broadcasted_iota(jnp.int32, (1, 1, _ROW), 2)
    for k in range(3):
        ck = np_ref[0, k, :]
        xg = xg - jnp.where(lane == k, jnp.float32(1.0),
                            jnp.float32(0.0)) * ck[:, None, None]
    x = xg.reshape(_CBLK * _NS, _ROW)
    dot = functools.partial(jnp.dot, preferred_element_type=jnp.float32,
                            precision=lax.Precision.HIGHEST)
    h = jnp.maximum(dot(x, w1_ref[...]) + b1_ref[...], 0.0)
    h = jnp.maximum(dot(h, w2_ref[...]) + b2_ref[...], 0.0)
    h = jnp.maximum(dot(h, w3_ref[...]) + b3_ref[...], 0.0)
    h = h.reshape(_CBLK, _NS, 64)
    slot = lax.broadcasted_iota(jnp.int32, (_CBLK, _NS, 64), 1)
    cnt = cnt_ref[0, 0].astype(jnp.int32)
    h = jnp.where(slot < cnt[:, None, None], h, jnp.float32(0.0))
    out_ref[0] = jnp.max(h, axis=1)


def _mlp(grouped, newp, cntf, w1e, b1r, w2t, b2r, w3t, b3r):
    nblk = _NP // _CBLK
    return pl.pallas_call(
        _mlp_body,
        grid=(_B, nblk),
        in_specs=[
            pl.BlockSpec((1, _CBLK, _NS, _ROW), lambda b, c: (b, c, 0, 0)),
            pl.BlockSpec((1, 3, _CBLK), lambda b, c: (b, 0, c)),
            pl.BlockSpec((1, 1, _CBLK), lambda b, c: (b * 8 + c, 0, 0)),
            pl.BlockSpec((_ROW, 32), lambda b, c: (0, 0)),
            pl.BlockSpec((1, 32), lambda b, c: (0, 0)),
            pl.BlockSpec((32, 32), lambda b, c: (0, 0)),
            pl.BlockSpec((1, 32), lambda b, c: (0, 0)),
            pl.BlockSpec((32, 64), lambda b, c: (0, 0)),
            pl.BlockSpec((1, 64), lambda b, c: (0, 0)),
        ],
        out_specs=pl.BlockSpec((1, _CBLK, 64), lambda b, c: (b, c, 0)),
        out_shape=jax.ShapeDtypeStruct((_B, _NP, 64), jnp.float32),
    )(grouped, newp, cntf, w1e, b1r, w2t, b2r, w3t, b3r)


def kernel(xyz, features, W1, b1, W2, b2, W3, b3):
    xyzp = jnp.transpose(xyz, (0, 2, 1))
    newp = _fps(xyzp.reshape(_B, 3, 8, 1024))

    table = jnp.concatenate(
        [xyz, jnp.transpose(features, (0, 2, 1)),
         jnp.zeros((_B, _N, _ROW - 3 - _C), jnp.float32)],
        axis=-1).reshape(_B * _N, _ROW)
    grouped, cntf = _bq(xyzp.reshape(_B * 3, _N), newp.reshape(_B * 3, _NP),
                        table)

    w1e = jnp.concatenate(
        [W1, jnp.zeros((32, _ROW - 3 - _C), jnp.float32)], axis=1).T
    out = _mlp(grouped, newp, cntf, w1e, b1.reshape(1, 32),
               W2.T, b2.reshape(1, 32), W3.T, b3.reshape(1, 64))

    new_xyz = jnp.transpose(newp, (0, 2, 1))
    new_features = jnp.transpose(out, (0, 2, 1))
    return (new_xyz, new_features)


# speedup vs baseline: 7.9929x; 1.0703x over previous
import functools

import jax
import jax.numpy as jnp
import numpy as np
from jax import lax
from jax.experimental import pallas as pl
from jax.experimental.pallas import tpu as pltpu
from jax.experimental.pallas import tpu_sc as plsc

_B, _N, _C = 4, 8192, 16
_NP, _NS = 1024, 32
_R2 = float(np.float32(0.8 * 0.8))
_ROW = 24
_CBLK = 128


def _fps_body(xyz_ref, xyzs_ref, new_ref):
    x = xyz_ref[0, 0]
    y = xyz_ref[0, 1]
    z = xyz_ref[0, 2]
    ii = (lax.broadcasted_iota(jnp.int32, (8, 1024), 0) * 1024
          + lax.broadcasted_iota(jnp.int32, (8, 1024), 1))

    def coords(p):
        return (xyzs_ref[0, 0, p], xyzs_ref[0, 1, p], xyzs_ref[0, 2, p])

    def body(i, st):
        dists, p = st
        lx, ly, lz = coords(p)
        new_ref[0, 0, i - 1] = lx
        new_ref[0, 1, i - 1] = ly
        new_ref[0, 2, i - 1] = lz
        dx = x - lx
        dy = y - ly
        dz = z - lz
        d = (dx * dx + dy * dy) + dz * dz
        dists = jnp.minimum(dists, d)
        m = jnp.max(dists)
        cand = jnp.where(dists == m, ii, _N)
        return dists, jnp.min(cand)

    dists0 = jnp.full((8, 1024), 1e10, jnp.float32)
    _, p = lax.fori_loop(1, _NP, body, (dists0, jnp.int32(0)))
    lx, ly, lz = coords(p)
    new_ref[0, 0, _NP - 1] = lx
    new_ref[0, 1, _NP - 1] = ly
    new_ref[0, 2, _NP - 1] = lz


def _fps(xyz_r, xyzp):
    return pl.pallas_call(
        _fps_body,
        grid=(_B,),
        in_specs=[
            pl.BlockSpec((1, 3, 8, 1024), lambda b: (b, 0, 0, 0)),
            pl.BlockSpec((1, 3, _N), lambda b: (b, 0, 0),
                         memory_space=pltpu.SMEM),
        ],
        out_specs=pl.BlockSpec((1, 3, _NP), lambda b: (b, 0, 0),
                               memory_space=pltpu.SMEM),
        out_shape=jax.ShapeDtypeStruct((_B, 3, _NP), jnp.float32),
    )(xyz_r, xyzp)


def _bq_body(xyzp, newp, table, grouped, cntf,
             px, py, pz, cx, cy, cz, idxbuf, idx32, rows, cntb, sem):
    w = lax.axis_index("s") * 2 + lax.axis_index("c")
    b = w // 8
    c0 = (w % 8) * _CBLK

    pltpu.sync_copy(xyzp.at[b * 3 + 0], px)
    pltpu.sync_copy(xyzp.at[b * 3 + 1], py)
    pltpu.sync_copy(xyzp.at[b * 3 + 2], pz)
    pltpu.sync_copy(newp.at[b * 3 + 0], cx)
    pltpu.sync_copy(newp.at[b * 3 + 1], cy)
    pltpu.sync_copy(newp.at[b * 3 + 2], cz)

    zeros16 = jnp.zeros((16,), jnp.int32)
    lanes = lax.iota(jnp.int32, 16)
    bofs = jnp.full((16,), b * _N, jnp.int32)

    def do_centroid(j, carry):
        cg = c0 + j
        csel = jnp.full((16,), cg, jnp.int32)
        cxv = plsc.load_gather(cx, [csel])
        cyv = plsc.load_gather(cy, [csel])
        czv = plsc.load_gather(cz, [csel])
        idxbuf[pl.ds(0, 16)] = zeros16
        idxbuf[pl.ds(16, 16)] = zeros16
        idxbuf[pl.ds(32, 16)] = zeros16

        def scan_cond(st):
            i, cnt = st
            return jnp.logical_and(i < _N // 16, cnt < _NS)

        def scan(st):
            i, cnt = st
            base = i * 16
            dx = px[pl.ds(base, 16)] - cxv
            dy = py[pl.ds(base, 16)] - cyv
            dz = pz[pl.ds(base, 16)] - czv
            d2 = (dx * dx + dy * dy) + dz * dz
            within = d2 < _R2
            plsc.store_compressed(idxbuf.at[pl.ds(cnt, 16)],
                                  lanes + base, mask=within)
            inc = jnp.sum(jnp.where(within, 1, 0).astype(jnp.int32))
            return i + 1, cnt + inc

        _, cnt = lax.while_loop(scan_cond, scan,
                                (jnp.int32(0), jnp.int32(0)))
        cnt = jnp.minimum(cnt, _NS)

        idx32[pl.ds(0, 16)] = idxbuf[pl.ds(0, 16)] + bofs
        idx32[pl.ds(16, 16)] = idxbuf[pl.ds(16, 16)] + bofs
        pltpu.async_copy(table.at[idx32], rows, sem).wait()
        pltpu.sync_copy(rows, grouped.at[b, cg])
        plsc.store_scatter(cntb, [jnp.full((16,), j, jnp.int32)],
                           jnp.full((16,), cnt.astype(jnp.float32)))
        return carry

    lax.fori_loop(0, _CBLK, do_centroid, jnp.int32(0))
    pltpu.sync_copy(cntb, cntf.at[w, 0])


_bq_cache = []


def _bq(*args):
    if not _bq_cache:
        _bq_cache.append(_make_bq())
    return _bq_cache[0](*args)


def _make_bq():
    return functools.partial(
        pl.kernel,
        mesh=plsc.VectorSubcoreMesh(core_axis_name="c", subcore_axis_name="s"),
        compiler_params=pltpu.CompilerParams(needs_layout_passes=False,
                                             use_tc_tiling_on_sc=False),
        out_type=(jax.ShapeDtypeStruct((_B, _NP, _NS, _ROW), jnp.float32),
                  jax.ShapeDtypeStruct((_B * 8, 1, _CBLK), jnp.float32)),
        scratch_types=[
        pltpu.VMEM((_N,), jnp.float32),
        pltpu.VMEM((_N,), jnp.float32),
        pltpu.VMEM((_N,), jnp.float32),
        pltpu.VMEM((_NP,), jnp.float32),
        pltpu.VMEM((_NP,), jnp.float32),
        pltpu.VMEM((_NP,), jnp.float32),
        pltpu.VMEM((48,), jnp.int32),
        pltpu.VMEM((_NS,), jnp.int32),
            pltpu.VMEM((_NS, _ROW), jnp.float32),
            pltpu.VMEM((_CBLK,), jnp.float32),
            pltpu.SemaphoreType.DMA,
        ],
    )(_bq_body)


def _mlp_body(g_ref, np_ref, cnt_ref, w1_ref, b1_ref, w2_ref, b2_ref,
              w3_ref, b3_ref, out_ref):
    xg = g_ref[0]
    lane = lax.broadcasted_iota(jnp.int32, (1, 1, _ROW), 2)
    for k in range(3):
        ck = np_ref[0, k, :]
        xg = xg - jnp.where(lane == k, jnp.float32(1.0),
                            jnp.float32(0.0)) * ck[:, None, None]
    x = xg.reshape(_CBLK * _NS, _ROW)
    dot = functools.partial(jnp.dot, preferred_element_type=jnp.float32,
                            precision=lax.Precision.HIGHEST)
    h = jnp.maximum(dot(x, w1_ref[...]) + b1_ref[...], 0.0)
    h = jnp.maximum(dot(h, w2_ref[...]) + b2_ref[...], 0.0)
    h = jnp.maximum(dot(h, w3_ref[...]) + b3_ref[...], 0.0)
    h = h.reshape(_CBLK, _NS, 64)
    slot = lax.broadcasted_iota(jnp.int32, (_CBLK, _NS, 64), 1)
    cnt = cnt_ref[0, 0].astype(jnp.int32)
    h = jnp.where(slot < cnt[:, None, None], h, jnp.float32(0.0))
    out_ref[0] = jnp.max(h, axis=1)


def _mlp(grouped, newp, cntf, w1e, b1r, w2t, b2r, w3t, b3r):
    nblk = _NP // _CBLK
    return pl.pallas_call(
        _mlp_body,
        grid=(_B, nblk),
        in_specs=[
            pl.BlockSpec((1, _CBLK, _NS, _ROW), lambda b, c: (b, c, 0, 0)),
            pl.BlockSpec((1, 3, _CBLK), lambda b, c: (b, 0, c)),
            pl.BlockSpec((1, 1, _CBLK), lambda b, c: (b * 8 + c, 0, 0)),
            pl.BlockSpec((_ROW, 32), lambda b, c: (0, 0)),
            pl.BlockSpec((1, 32), lambda b, c: (0, 0)),
            pl.BlockSpec((32, 32), lambda b, c: (0, 0)),
            pl.BlockSpec((1, 32), lambda b, c: (0, 0)),
            pl.BlockSpec((32, 64), lambda b, c: (0, 0)),
            pl.BlockSpec((1, 64), lambda b, c: (0, 0)),
        ],
        out_specs=pl.BlockSpec((1, _CBLK, 64), lambda b, c: (b, c, 0)),
        out_shape=jax.ShapeDtypeStruct((_B, _NP, 64), jnp.float32),
    )(grouped, newp, cntf, w1e, b1r, w2t, b2r, w3t, b3r)


def kernel(xyz, features, W1, b1, W2, b2, W3, b3):
    xyzp = jnp.transpose(xyz, (0, 2, 1))
    newp = _fps(xyzp.reshape(_B, 3, 8, 1024), xyzp)

    table = jnp.concatenate(
        [xyz, jnp.transpose(features, (0, 2, 1)),
         jnp.zeros((_B, _N, _ROW - 3 - _C), jnp.float32)],
        axis=-1).reshape(_B * _N, _ROW)
    grouped, cntf = _bq(xyzp.reshape(_B * 3, _N), newp.reshape(_B * 3, _NP),
                        table)

    w1e = jnp.concatenate(
        [W1, jnp.zeros((32, _ROW - 3 - _C), jnp.float32)], axis=1).T
    out = _mlp(grouped, newp, cntf, w1e, b1.reshape(1, 32),
               W2.T, b2.reshape(1, 32), W3.T, b3.reshape(1, 64))

    new_xyz = jnp.transpose(newp, (0, 2, 1))
    new_features = jnp.transpose(out, (0, 2, 1))
    return (new_xyz, new_features)


# speedup vs baseline: 10.1876x; 1.2746x over previous
import functools

import jax
import jax.numpy as jnp
import numpy as np
from jax import lax
from jax.experimental import pallas as pl
from jax.experimental.pallas import tpu as pltpu
from jax.experimental.pallas import tpu_sc as plsc

_B, _N, _C = 4, 8192, 16
_NP, _NS = 1024, 32
_R2 = float(np.float32(0.8 * 0.8))
_ROW = 24
_CBLK = 128


def _fps_body(xyz_ref, xyzs_ref, new_ref):
    x = xyz_ref[0, 0]
    y = xyz_ref[0, 1]
    z = xyz_ref[0, 2]
    ii = (lax.broadcasted_iota(jnp.int32, (8, 1024), 0) * 1024
          + lax.broadcasted_iota(jnp.int32, (8, 1024), 1))

    def coords(p):
        return (xyzs_ref[0, 0, p], xyzs_ref[0, 1, p], xyzs_ref[0, 2, p])

    def body(i, st):
        dists, p = st
        lx, ly, lz = coords(p)
        new_ref[0, 0, i - 1] = lx
        new_ref[0, 1, i - 1] = ly
        new_ref[0, 2, i - 1] = lz
        dx = x - lx
        dy = y - ly
        dz = z - lz
        d = (dx * dx + dy * dy) + dz * dz
        dists = jnp.minimum(dists, d)
        m = jnp.max(dists)
        cand = jnp.where(dists == m, ii, _N)
        return dists, jnp.min(cand)

    dists0 = jnp.full((8, 1024), 1e10, jnp.float32)
    _, p = lax.fori_loop(1, _NP, body, (dists0, jnp.int32(0)))
    lx, ly, lz = coords(p)
    new_ref[0, 0, _NP - 1] = lx
    new_ref[0, 1, _NP - 1] = ly
    new_ref[0, 2, _NP - 1] = lz


def _fps(xyz_r, xyzp):
    return pl.pallas_call(
        _fps_body,
        grid=(_B,),
        in_specs=[
            pl.BlockSpec((1, 3, 8, 1024), lambda b: (b, 0, 0, 0)),
            pl.BlockSpec((1, 3, _N), lambda b: (b, 0, 0),
                         memory_space=pltpu.SMEM),
        ],
        out_specs=pl.BlockSpec((1, 3, _NP), lambda b: (b, 0, 0),
                               memory_space=pltpu.SMEM),
        out_shape=jax.ShapeDtypeStruct((_B, 3, _NP), jnp.float32),
    )(xyz_r, xyzp)


def _bq_body(xyzp, newp, table, grouped, cntf,
             px, py, pz, cx, cy, cz, idxbuf, idx32, rows, cntb, sem):
    w = lax.axis_index("s") * 2 + lax.axis_index("c")
    b = w // 8
    c0 = (w % 8) * _CBLK

    pltpu.sync_copy(xyzp.at[b * 3 + 0], px)
    pltpu.sync_copy(xyzp.at[b * 3 + 1], py)
    pltpu.sync_copy(xyzp.at[b * 3 + 2], pz)
    pltpu.sync_copy(newp.at[b * 3 + 0], cx)
    pltpu.sync_copy(newp.at[b * 3 + 1], cy)
    pltpu.sync_copy(newp.at[b * 3 + 2], cz)

    zeros16 = jnp.zeros((16,), jnp.int32)
    lanes = lax.iota(jnp.int32, 16)
    bofs = jnp.full((16,), b * _N, jnp.int32)

    def do_centroid(j, carry):
        cg = c0 + j
        csel = jnp.full((16,), cg, jnp.int32)
        cxv = plsc.load_gather(cx, [csel])
        cyv = plsc.load_gather(cy, [csel])
        czv = plsc.load_gather(cz, [csel])
        idxbuf[pl.ds(0, 16)] = zeros16
        idxbuf[pl.ds(16, 16)] = zeros16
        idxbuf[pl.ds(32, 16)] = zeros16

        def scan(k, st):
            s, cnt = st
            base = s * 1024 + k * 16
            dx = px[pl.ds(base, 16)] - cxv
            dy = py[pl.ds(base, 16)] - cyv
            dz = pz[pl.ds(base, 16)] - czv
            d2 = (dx * dx + dy * dy) + dz * dz
            within = d2 < _R2
            plsc.store_compressed(idxbuf.at[pl.ds(jnp.minimum(cnt, _NS), 16)],
                                  lanes + base, mask=within)
            inc = jnp.sum(jnp.where(within, 1, 0).astype(jnp.int32))
            return s, cnt + inc

        def super_cond(st):
            s, cnt = st
            return jnp.logical_and(s < 8, cnt < _NS)

        def super_body(st):
            return lax.fori_loop(0, 64, scan, (st[0], st[1]))

        def super_step(st):
            s, cnt = super_body(st)
            return s + 1, cnt

        _, cnt = lax.while_loop(super_cond, super_step,
                                (jnp.int32(0), jnp.int32(0)))
        cnt = jnp.minimum(cnt, _NS)

        idx32[pl.ds(0, 16)] = idxbuf[pl.ds(0, 16)] + bofs
        idx32[pl.ds(16, 16)] = idxbuf[pl.ds(16, 16)] + bofs
        pltpu.async_copy(table.at[idx32], rows, sem).wait()
        pltpu.sync_copy(rows, grouped.at[b, cg])
        plsc.store_scatter(cntb, [jnp.full((16,), j, jnp.int32)],
                           jnp.full((16,), cnt.astype(jnp.float32)))
        return carry

    lax.fori_loop(0, _CBLK, do_centroid, jnp.int32(0))
    pltpu.sync_copy(cntb, cntf.at[w, 0])


_bq_cache = []


def _bq(*args):
    if not _bq_cache:
        _bq_cache.append(_make_bq())
    return _bq_cache[0](*args)


def _make_bq():
    return functools.partial(
        pl.kernel,
        mesh=plsc.VectorSubcoreMesh(core_axis_name="c", subcore_axis_name="s"),
        compiler_params=pltpu.CompilerParams(needs_layout_passes=False,
                                             use_tc_tiling_on_sc=False),
        out_type=(jax.ShapeDtypeStruct((_B, _NP, _NS, _ROW), jnp.float32),
                  jax.ShapeDtypeStruct((_B * 8, 1, _CBLK), jnp.float32)),
        scratch_types=[
        pltpu.VMEM((_N,), jnp.float32),
        pltpu.VMEM((_N,), jnp.float32),
        pltpu.VMEM((_N,), jnp.float32),
        pltpu.VMEM((_NP,), jnp.float32),
        pltpu.VMEM((_NP,), jnp.float32),
        pltpu.VMEM((_NP,), jnp.float32),
        pltpu.VMEM((48,), jnp.int32),
        pltpu.VMEM((_NS,), jnp.int32),
            pltpu.VMEM((_NS, _ROW), jnp.float32),
            pltpu.VMEM((_CBLK,), jnp.float32),
            pltpu.SemaphoreType.DMA,
        ],
    )(_bq_body)


def _mlp_body(g_ref, np_ref, cnt_ref, w1_ref, b1_ref, w2_ref, b2_ref,
              w3_ref, b3_ref, out_ref):
    xg = g_ref[0]
    lane = lax.broadcasted_iota(jnp.int32, (1, 1, _ROW), 2)
    for k in range(3):
        ck = np_ref[0, k, :]
        xg = xg - jnp.where(lane == k, jnp.float32(1.0),
                            jnp.float32(0.0)) * ck[:, None, None]
    x = xg.reshape(_CBLK * _NS, _ROW)
    dot = functools.partial(jnp.dot, preferred_element_type=jnp.float32,
                            precision=lax.Precision.HIGHEST)
    h = jnp.maximum(dot(x, w1_ref[...]) + b1_ref[...], 0.0)
    h = jnp.maximum(dot(h, w2_ref[...]) + b2_ref[...], 0.0)
    h = jnp.maximum(dot(h, w3_ref[...]) + b3_ref[...], 0.0)
    h = h.reshape(_CBLK, _NS, 64)
    slot = lax.broadcasted_iota(jnp.int32, (_CBLK, _NS, 64), 1)
    cnt = cnt_ref[0, 0].astype(jnp.int32)
    h = jnp.where(slot < cnt[:, None, None], h, jnp.float32(0.0))
    out_ref[0] = jnp.max(h, axis=1)


def _mlp(grouped, newp, cntf, w1e, b1r, w2t, b2r, w3t, b3r):
    nblk = _NP // _CBLK
    return pl.pallas_call(
        _mlp_body,
        grid=(_B, nblk),
        in_specs=[
            pl.BlockSpec((1, _CBLK, _NS, _ROW), lambda b, c: (b, c, 0, 0)),
            pl.BlockSpec((1, 3, _CBLK), lambda b, c: (b, 0, c)),
            pl.BlockSpec((1, 1, _CBLK), lambda b, c: (b * 8 + c, 0, 0)),
            pl.BlockSpec((_ROW, 32), lambda b, c: (0, 0)),
            pl.BlockSpec((1, 32), lambda b, c: (0, 0)),
            pl.BlockSpec((32, 32), lambda b, c: (0, 0)),
            pl.BlockSpec((1, 32), lambda b, c: (0, 0)),
            pl.BlockSpec((32, 64), lambda b, c: (0, 0)),
            pl.BlockSpec((1, 64), lambda b, c: (0, 0)),
        ],
        out_specs=pl.BlockSpec((1, _CBLK, 64), lambda b, c: (b, c, 0)),
        out_shape=jax.ShapeDtypeStruct((_B, _NP, 64), jnp.float32),
    )(grouped, newp, cntf, w1e, b1r, w2t, b2r, w3t, b3r)


def kernel(xyz, features, W1, b1, W2, b2, W3, b3):
    xyzp = jnp.transpose(xyz, (0, 2, 1))
    newp = _fps(xyzp.reshape(_B, 3, 8, 1024), xyzp)

    table = jnp.concatenate(
        [xyz, jnp.transpose(features, (0, 2, 1)),
         jnp.zeros((_B, _N, _ROW - 3 - _C), jnp.float32)],
        axis=-1).reshape(_B * _N, _ROW)
    grouped, cntf = _bq(xyzp.reshape(_B * 3, _N), newp.reshape(_B * 3, _NP),
                        table)

    w1e = jnp.concatenate(
        [W1, jnp.zeros((32, _ROW - 3 - _C), jnp.float32)], axis=1).T
    out = _mlp(grouped, newp, cntf, w1e, b1.reshape(1, 32),
               W2.T, b2.reshape(1, 32), W3.T, b3.reshape(1, 64))

    new_xyz = jnp.transpose(newp, (0, 2, 1))
    new_features = jnp.transpose(out, (0, 2, 1))
    return (new_xyz, new_features)
